# baseline (speedup 1.0000x reference)
import functools
import jax
import jax.numpy as jnp
from jax import lax
from jax.experimental import pallas as pl
from jax.experimental.pallas import tpu as pltpu
from jax.experimental.pallas import tpu_sc as plsc

N_DATA = 1_000_000
N_OLD = 100_000
M_TOP = 1000
NW = 16
PW = 62_528
NV = PW // 16
N_PAD = NW * PW
NB = 1024
CAP_L = 512
CAP_F = 256
SRT = 1536
SEL = 1024
OUT_PAD = N_OLD + SEL
INT_MIN = jnp.int32(-2147483648)
Q1 = N_PAD - M_TOP

_mesh = plsc.VectorSubcoreMesh(core_axis_name="c", subcore_axis_name="s")


def _key_of(x):
    s = plsc.bitcast(x, jnp.int32)
    return jnp.where(s >= 0, s, INT_MIN - s)


def _butterfly_sum(v):
    iota = lax.iota(jnp.int32, 16)
    for k in (8, 4, 2, 1):
        v = v + v.at[jnp.bitwise_xor(iota, k)].get(mode="promise_in_bounds")
    return v


def _zero_hist(hist):
    iota = lax.iota(jnp.int32, 16)
    zeros = jnp.zeros((16,), jnp.int32)

    def body(i, _):
        plsc.store_scatter(hist, [i * 16 + iota], zeros)
        return 0

    lax.fori_loop(0, NB, body, 0)


def _merge_hists(w, hist, merged, sh_hist, sh_merged):
    iota = lax.iota(jnp.int32, 16)
    pltpu.sync_copy(hist, sh_hist.at[pl.ds(w * (NB * 16), NB * 16)])
    plsc.subcore_barrier()
    for r in range(NW):
        pltpu.sync_copy(
            sh_hist.at[pl.ds(r * (NB * 16) + w * NB, NB)],
            hist.at[pl.ds(r * NB, NB)],
        )

    def vsum(i, _):
        acc = jnp.zeros((16,), jnp.int32)
        for r in range(NW):
            acc = acc + plsc.load_gather(hist, [r * NB + i * 16 + iota])
        plsc.store_scatter(hist, [i * 16 + iota], acc)
        return 0

    lax.fori_loop(0, NB // 16, vsum, 0)

    def lsum(b, _):
        v = plsc.load_gather(hist, [b * 16 + iota])
        tot = _butterfly_sum(v)
        plsc.store_scatter(merged, [jnp.full((16,), b, jnp.int32)], tot,
                           mask=iota == 0)
        return 0

    lax.fori_loop(0, NB // 16, lsum, 0)
    pltpu.sync_copy(merged.at[pl.ds(0, NB // 16)], sh_merged.at[pl.ds(w * (NB // 16), NB // 16)])
    plsc.subcore_barrier()
    pltpu.sync_copy(sh_merged, merged)


def _find_crossing(merged, q):
    iota = lax.iota(jnp.int32, 16)

    def body(v, carry):
        run, bstar, cbefore = carry
        h = plsc.load_gather(merged, [v * 16 + iota])
        cs = plsc.cumsum(h)
        tot = jnp.max(cs)
        crossed = (run + cs) > q
        ffs = jnp.max(plsc.all_reduce_ffs(crossed).astype(jnp.int32))
        found = (bstar < 0) & (ffs < 16)
        excl = cs - h
        cb = run + jnp.max(jnp.where(iota == ffs, excl, 0))
        bstar = jnp.where(found, v * 16 + ffs, bstar)
        cbefore = jnp.where(found, cb, cbefore)
        return run + tot, bstar, cbefore

    _, bstar, cbefore = lax.fori_loop(
        0, NB // 16, body, (jnp.int32(0), jnp.int32(-1), jnp.int32(0)))
    return bstar, cbefore


@functools.partial(
    pl.kernel,
    out_type=[
        jax.ShapeDtypeStruct((OUT_PAD,), jnp.float32),
        jax.ShapeDtypeStruct((OUT_PAD,), jnp.float32),
    ],
    mesh=_mesh,
    compiler_params=pltpu.CompilerParams(needs_layout_passes=False),
    scratch_types=[
        pltpu.VMEM((PW,), jnp.float32),
        pltpu.VMEM((NB * 16,), jnp.int32),
        pltpu.VMEM((NB,), jnp.int32),
        pltpu.VMEM((CAP_L,), jnp.int32),
        pltpu.VMEM((CAP_L,), jnp.int32),
        pltpu.VMEM((CAP_F,), jnp.int32),
        pltpu.VMEM((CAP_F,), jnp.int32),
        pltpu.VMEM((NW * 16,), jnp.int32),
        pltpu.VMEM((NW * CAP_F,), jnp.int32),
        pltpu.VMEM((NW * CAP_F,), jnp.int32),
        pltpu.VMEM((SRT,), jnp.int32),
        pltpu.VMEM((SRT,), jnp.int32),
        pltpu.VMEM((SRT,), jnp.int32),
        pltpu.VMEM((SRT,), jnp.int32),
        pltpu.VMEM((128,), jnp.int32),
        pltpu.VMEM((64,), jnp.int32),
        pltpu.VMEM((64,), jnp.float32),
        pltpu.VMEM((64,), jnp.float32),
        pltpu.VMEM((16,), jnp.int32),
        pltpu.VMEM_SHARED((NW * NB * 16,), jnp.int32),
        pltpu.VMEM_SHARED((NB,), jnp.int32),
        pltpu.VMEM_SHARED((NW * CAP_F,), jnp.int32),
        pltpu.VMEM_SHARED((NW * CAP_F,), jnp.int32),
        pltpu.VMEM_SHARED((NW * 16,), jnp.int32),
        pltpu.VMEM_SHARED((SEL,), jnp.int32),
        pltpu.SemaphoreType.DMA,
    ],
)
def _sc_topk(loss_hbm, xf_hbm, tf_hbm, xs_hbm, ts_hbm, outx_hbm, outt_hbm,
             data_v, hist_v, merged_v, ck_v, cg_v, fck_v, fcg_v, cnts_v,
             allk_v, allg_v, ka, ga, kb, gb, offs_v, sel_v, rx_v, rt_v, v16,
             sh_hist, sh_merged, sh_fck, sh_fcg, sh_fcnt, sh_sel, sem):
    c = lax.axis_index("c")
    w = lax.axis_index("s")
    iota = lax.iota(jnp.int32, 16)
    ones = jnp.ones((16,), jnp.int32)

    @pl.when(c == 1)
    def _copy_old():
        @pl.when(w < 12)
        def _():
            a = w * 6248
            pltpu.sync_copy(xf_hbm.at[pl.ds(a, 6248)], data_v.at[pl.ds(0, 6248)])
            pltpu.sync_copy(data_v.at[pl.ds(0, 6248)], outx_hbm.at[pl.ds(a, 6248)])
            pltpu.sync_copy(tf_hbm.at[pl.ds(a, 6248)], data_v.at[pl.ds(0, 6248)])
            pltpu.sync_copy(data_v.at[pl.ds(0, 6248)], outt_hbm.at[pl.ds(a, 6248)])

        @pl.when(w >= 12)
        def _():
            a = 74976 + (w - 12) * 6256
            pltpu.sync_copy(xf_hbm.at[pl.ds(a, 6256)], data_v.at[pl.ds(0, 6256)])
            pltpu.sync_copy(data_v.at[pl.ds(0, 6256)], outx_hbm.at[pl.ds(a, 6256)])
            pltpu.sync_copy(tf_hbm.at[pl.ds(a, 6256)], data_v.at[pl.ds(0, 6256)])
            pltpu.sync_copy(data_v.at[pl.ds(0, 6256)], outt_hbm.at[pl.ds(a, 6256)])

    @pl.when(c == 0)
    def _topk():
        pltpu.sync_copy(loss_hbm.at[pl.ds(w * PW, PW)], data_v)
        _zero_hist(hist_v)

        def hist_body(i, _):
            x = data_v[pl.ds(i * 16, 16)]
            key = _key_of(x)
            b1 = jnp.right_shift(key, 22) + 512
            plsc.addupdate_scatter(hist_v, [b1 * 16 + iota], ones)
            return 0

        lax.fori_loop(0, NV, hist_body, 0)
        _merge_hists(w, hist_v, merged_v, sh_hist, sh_merged)
        b1star, cbefore1 = _find_crossing(merged_v, Q1)

        def compact_body(i, cnt):
            x = data_v[pl.ds(i * 16, 16)]
            key = _key_of(x)
            b1 = jnp.right_shift(key, 22) + 512
            gidx = w * PW + i * 16 + iota
            m = (b1 >= b1star) & (gidx < N_DATA)

            def emit(cnt, m=m, key=key, gidx=gidx):
                rcs = plsc.cumsum(m.astype(jnp.int32))
                pos = cnt + rcs - 1
                m2 = m & (pos < CAP_L)
                plsc.store_scatter(ck_v, [pos], key, mask=m2)
                plsc.store_scatter(cg_v, [pos], gidx, mask=m2)
                return cnt + jnp.max(rcs)

            return lax.cond(jnp.any(m), emit, lambda c: c, cnt)

        cnt = lax.fori_loop(0, NV, compact_body, jnp.int32(0))
        cnt = jnp.minimum(cnt, CAP_L)

        _zero_hist(hist_v)

        def hist2_body(i, _):
            kv = plsc.load_gather(ck_v, [i * 16 + iota])
            valid = (i * 16 + iota) < cnt
            in_bucket = (jnp.right_shift(kv, 22) + 512) == b1star
            b2 = jnp.bitwise_and(jnp.right_shift(kv, 12), NB - 1)
            plsc.addupdate_scatter(hist_v, [b2 * 16 + iota], ones,
                                   mask=valid & in_bucket)
            return 0

        lax.fori_loop(0, (cnt + 15) // 16, hist2_body, 0)
        _merge_hists(w, hist_v, merged_v, sh_hist, sh_merged)
        b2star, _cb2 = _find_crossing(merged_v, Q1 - cbefore1)
        t22 = (b1star - 512) * NB + b2star

        def filt_body(i, fcnt):
            kv = plsc.load_gather(ck_v, [i * 16 + iota])
            gv = plsc.load_gather(cg_v, [i * 16 + iota])
            valid = (i * 16 + iota) < cnt
            m = valid & (jnp.right_shift(kv, 12) >= t22)
            rcs = plsc.cumsum(m.astype(jnp.int32))
            pos = fcnt + rcs - 1
            m2 = m & (pos < CAP_F)
            plsc.store_scatter(fck_v, [pos], kv, mask=m2)
            plsc.store_scatter(fcg_v, [pos], gv, mask=m2)
            return fcnt + jnp.max(rcs)

        fcnt = lax.fori_loop(0, (cnt + 15) // 16, filt_body, jnp.int32(0))
        fcnt = jnp.minimum(fcnt, CAP_F)
        pltpu.sync_copy(fck_v, sh_fck.at[pl.ds(w * CAP_F, CAP_F)])
        pltpu.sync_copy(fcg_v, sh_fcg.at[pl.ds(w * CAP_F, CAP_F)])
        v16[...] = jnp.full((16,), fcnt, jnp.int32)
        pltpu.sync_copy(v16, sh_fcnt.at[pl.ds(w * 16, 16)])
        plsc.subcore_barrier()

        @pl.when(w == 0)
        def _sort():
            pltpu.sync_copy(sh_fck, allk_v)
            pltpu.sync_copy(sh_fcg, allg_v)
            pltpu.sync_copy(sh_fcnt, cnts_v)
            neg = jnp.full((16,), INT_MIN, jnp.int32)
            zer = jnp.zeros((16,), jnp.int32)
            for i in range(SRT // 16):
                ka[pl.ds(i * 16, 16)] = neg
                ga[pl.ds(i * 16, 16)] = zer

            tail = jnp.int32(0)
            for w2 in range(NW):
                cw = jnp.max(cnts_v[pl.ds(w2 * 16, 16)])

                def asm_body(v, tl, w2=w2, cw=cw):
                    kv = plsc.load_gather(allk_v, [w2 * CAP_F + v * 16 + iota])
                    gv = plsc.load_gather(allg_v, [w2 * CAP_F + v * 16 + iota])
                    m = (v * 16 + iota) < cw
                    pos = tl + v * 16 + iota
                    m = m & (pos < SRT)
                    plsc.store_scatter(ka, [pos], kv, mask=m)
                    plsc.store_scatter(ga, [pos], gv, mask=m)
                    return tl

                lax.fori_loop(0, (cw + 15) // 16, asm_body, tail)
                tail = jnp.minimum(tail + cw, SRT)

            src_k, src_g, dst_k, dst_g = ka, ga, kb, gb
            for p in range(5):
                shift = 7 * p
                def zh(i, _):
                    plsc.store_scatter(hist_v, [i * 16 + iota], zer)
                    return 0

                lax.fori_loop(0, 128, zh, 0)

                def ph_a(i, _, src_k=src_k, shift=shift):
                    kv = plsc.load_gather(src_k, [i * 16 + iota])
                    uk = jnp.bitwise_xor(kv, INT_MIN)
                    d = 127 - jnp.bitwise_and(
                        lax.shift_right_logical(uk, shift), 127)
                    plsc.addupdate_scatter(hist_v, [d * 16 + iota], ones)
                    return 0

                lax.fori_loop(0, SRT // 16, ph_a, 0)

                run = jnp.int32(0)
                for g in range(8):
                    acc = zer
                    for l in range(16):
                        acc = acc + plsc.load_gather(
                            hist_v, [(g * 16 + iota) * 16 + l])
                    cs = plsc.cumsum(acc)
                    offs_v[pl.ds(g * 16, 16)] = run + cs - acc
                    run = run + jnp.max(cs)

                def ph_b(i, _, src_k=src_k, src_g=src_g, dst_k=dst_k,
                         dst_g=dst_g, shift=shift):
                    kv = plsc.load_gather(src_k, [i * 16 + iota])
                    gv = plsc.load_gather(src_g, [i * 16 + iota])
                    uk = jnp.bitwise_xor(kv, INT_MIN)
                    d = 127 - jnp.bitwise_and(
                        lax.shift_right_logical(uk, shift), 127)
                    comp = d * 16 + iota
                    sd, _sv = plsc.sort_key_val(comp, iota)
                    dso = jnp.right_shift(sd, 4)
                    lo = jnp.bitwise_and(sd, 15)
                    prev = dso.at[jnp.maximum(iota - 1, 0)].get(
                        mode="promise_in_bounds")
                    isstart = (iota == 0) | (dso != prev)
                    stl = plsc.cummax(jnp.where(isstart, iota, 0))
                    rc = iota - stl
                    nxt = dso.at[jnp.minimum(iota + 1, 15)].get(
                        mode="promise_in_bounds")
                    islast = (iota == 15) | (dso != nxt)
                    base = plsc.load_gather(offs_v, [dso])
                    pos = base + rc
                    kvs = kv.at[lo].get(mode="promise_in_bounds")
                    gvs = gv.at[lo].get(mode="promise_in_bounds")
                    plsc.store_scatter(dst_k, [pos], kvs)
                    plsc.store_scatter(dst_g, [pos], gvs)
                    plsc.addupdate_scatter(offs_v, [dso], rc + 1, mask=islast)
                    return 0

                lax.fori_loop(0, SRT // 16, ph_b, 0)
                src_k, src_g, dst_k, dst_g = dst_k, dst_g, src_k, src_g

            pltpu.sync_copy(src_g.at[pl.ds(0, SEL)], sh_sel)

        plsc.subcore_barrier()

        pltpu.sync_copy(sh_sel.at[pl.ds(w * 64, 64)], sel_v)
        pltpu.async_copy(xs_hbm.at[sel_v], rx_v, sem).wait()
        pltpu.async_copy(ts_hbm.at[sel_v], rt_v, sem).wait()
        pltpu.sync_copy(rx_v, outx_hbm.at[pl.ds(N_OLD + w * 64, 64)])
        pltpu.sync_copy(rt_v, outt_hbm.at[pl.ds(N_OLD + w * 64, 64)])


def kernel(loss, x_f, t_f, x_s, t_s):
    loss_p = jnp.concatenate(
        [loss.reshape(-1),
         jnp.full((N_PAD - N_DATA,), -jnp.inf, jnp.float32)])
    out_x, out_t = _sc_topk(loss_p, x_f.reshape(-1), t_f.reshape(-1),
                            x_s.reshape(-1), t_s.reshape(-1))
    return (out_x[:N_OLD + M_TOP].reshape(-1, 1),
            out_t[:N_OLD + M_TOP].reshape(-1, 1))


# speedup vs baseline: 1.1813x; 1.1813x over previous
import functools
import jax
import jax.numpy as jnp
from jax import lax
from jax.experimental import pallas as pl
from jax.experimental.pallas import tpu as pltpu
from jax.experimental.pallas import tpu_sc as plsc

N_DATA = 1_000_000
N_OLD = 100_000
M_TOP = 1000
NW = 16
PW = 62_528
NV = PW // 16
N_PAD = NW * PW
NB = 1024
CAP_L = 512
CAP_F = 256
SRT = 1536
SEL = 1024
OUT_PAD = N_OLD + SEL
INT_MIN = jnp.int32(-2147483648)
Q1 = N_PAD - M_TOP

_mesh = plsc.VectorSubcoreMesh(core_axis_name="c", subcore_axis_name="s")


def _key_of(x):
    s = plsc.bitcast(x, jnp.int32)
    return jnp.where(s >= 0, s, INT_MIN - s)


def _butterfly_sum(v):
    iota = lax.iota(jnp.int32, 16)
    for k in (8, 4, 2, 1):
        v = v + v.at[jnp.bitwise_xor(iota, k)].get(mode="promise_in_bounds")
    return v


def _zero_hist(hist):
    iota = lax.iota(jnp.int32, 16)
    zeros = jnp.zeros((16,), jnp.int32)

    def body(i, _):
        plsc.store_scatter(hist, [i * 16 + iota], zeros)
        return 0

    lax.fori_loop(0, NB, body, 0)


def _merge_hists(w, hist, merged, sh_hist, sh_merged):
    iota = lax.iota(jnp.int32, 16)
    pltpu.sync_copy(hist, sh_hist.at[pl.ds(w * (NB * 16), NB * 16)])
    plsc.subcore_barrier()
    for r in range(NW):
        pltpu.sync_copy(
            sh_hist.at[pl.ds(r * (NB * 16) + w * NB, NB)],
            hist.at[pl.ds(r * NB, NB)],
        )

    def vsum(i, _):
        acc = jnp.zeros((16,), jnp.int32)
        for r in range(NW):
            acc = acc + plsc.load_gather(hist, [r * NB + i * 16 + iota])
        plsc.store_scatter(hist, [i * 16 + iota], acc)
        return 0

    lax.fori_loop(0, NB // 16, vsum, 0)

    def lsum(b, _):
        v = plsc.load_gather(hist, [b * 16 + iota])
        tot = _butterfly_sum(v)
        plsc.store_scatter(merged, [jnp.full((16,), b, jnp.int32)], tot,
                           mask=iota == 0)
        return 0

    lax.fori_loop(0, NB // 16, lsum, 0)
    pltpu.sync_copy(merged.at[pl.ds(0, NB // 16)], sh_merged.at[pl.ds(w * (NB // 16), NB // 16)])
    plsc.subcore_barrier()
    pltpu.sync_copy(sh_merged, merged)


def _find_crossing(merged, q):
    iota = lax.iota(jnp.int32, 16)

    def body(v, carry):
        run, bstar, cbefore = carry
        h = plsc.load_gather(merged, [v * 16 + iota])
        cs = plsc.cumsum(h)
        tot = jnp.max(cs)
        crossed = (run + cs) > q
        ffs = jnp.max(plsc.all_reduce_ffs(crossed).astype(jnp.int32))
        found = (bstar < 0) & (ffs < 16)
        excl = cs - h
        cb = run + jnp.max(jnp.where(iota == ffs, excl, 0))
        bstar = jnp.where(found, v * 16 + ffs, bstar)
        cbefore = jnp.where(found, cb, cbefore)
        return run + tot, bstar, cbefore

    _, bstar, cbefore = lax.fori_loop(
        0, NB // 16, body, (jnp.int32(0), jnp.int32(-1), jnp.int32(0)))
    return bstar, cbefore


@functools.partial(
    pl.kernel,
    out_type=[
        jax.ShapeDtypeStruct((OUT_PAD,), jnp.float32),
        jax.ShapeDtypeStruct((OUT_PAD,), jnp.float32),
    ],
    mesh=_mesh,
    compiler_params=pltpu.CompilerParams(needs_layout_passes=False),
    scratch_types=[
        pltpu.VMEM((PW,), jnp.float32),
        pltpu.VMEM((NB * 16,), jnp.int32),
        pltpu.VMEM((NB,), jnp.int32),
        pltpu.VMEM((CAP_L,), jnp.int32),
        pltpu.VMEM((CAP_L,), jnp.int32),
        pltpu.VMEM((CAP_F,), jnp.int32),
        pltpu.VMEM((CAP_F,), jnp.int32),
        pltpu.VMEM((NW * 16,), jnp.int32),
        pltpu.VMEM((NW * CAP_F,), jnp.int32),
        pltpu.VMEM((NW * CAP_F,), jnp.int32),
        pltpu.VMEM((SRT,), jnp.int32),
        pltpu.VMEM((SRT,), jnp.int32),
        pltpu.VMEM((SRT,), jnp.int32),
        pltpu.VMEM((SRT,), jnp.int32),
        pltpu.VMEM((128,), jnp.int32),
        pltpu.VMEM((64,), jnp.int32),
        pltpu.VMEM((64,), jnp.float32),
        pltpu.VMEM((64,), jnp.float32),
        pltpu.VMEM((16,), jnp.int32),
        pltpu.VMEM_SHARED((NW * NB * 16,), jnp.int32),
        pltpu.VMEM_SHARED((NB,), jnp.int32),
        pltpu.VMEM_SHARED((NW * CAP_F,), jnp.int32),
        pltpu.VMEM_SHARED((NW * CAP_F,), jnp.int32),
        pltpu.VMEM_SHARED((NW * 16,), jnp.int32),
        pltpu.VMEM_SHARED((SEL,), jnp.int32),
        pltpu.SemaphoreType.DMA,
    ],
)
def _sc_topk(loss_hbm, xf_hbm, tf_hbm, xs_hbm, ts_hbm, outx_hbm, outt_hbm,
             data_v, hist_v, merged_v, ck_v, cg_v, fck_v, fcg_v, cnts_v,
             allk_v, allg_v, ka, ga, kb, gb, offs_v, sel_v, rx_v, rt_v, v16,
             sh_hist, sh_merged, sh_fck, sh_fcg, sh_fcnt, sh_sel, sem):
    c = lax.axis_index("c")
    w = lax.axis_index("s")
    iota = lax.iota(jnp.int32, 16)
    ones = jnp.ones((16,), jnp.int32)

    @pl.when(c == 1)
    def _copy_old():
        @pl.when(w < 12)
        def _():
            a = w * 6248
            pltpu.sync_copy(xf_hbm.at[pl.ds(a, 6248)], data_v.at[pl.ds(0, 6248)])
            pltpu.sync_copy(data_v.at[pl.ds(0, 6248)], outx_hbm.at[pl.ds(a, 6248)])
            pltpu.sync_copy(tf_hbm.at[pl.ds(a, 6248)], data_v.at[pl.ds(0, 6248)])
            pltpu.sync_copy(data_v.at[pl.ds(0, 6248)], outt_hbm.at[pl.ds(a, 6248)])

        @pl.when(w >= 12)
        def _():
            a = 74976 + (w - 12) * 6256
            pltpu.sync_copy(xf_hbm.at[pl.ds(a, 6256)], data_v.at[pl.ds(0, 6256)])
            pltpu.sync_copy(data_v.at[pl.ds(0, 6256)], outx_hbm.at[pl.ds(a, 6256)])
            pltpu.sync_copy(tf_hbm.at[pl.ds(a, 6256)], data_v.at[pl.ds(0, 6256)])
            pltpu.sync_copy(data_v.at[pl.ds(0, 6256)], outt_hbm.at[pl.ds(a, 6256)])

    @pl.when(c == 0)
    def _topk():
        pltpu.sync_copy(loss_hbm.at[pl.ds(w * PW, PW)], data_v)
        _zero_hist(hist_v)

        def hist_body(i, _):
            x = data_v[pl.ds(i * 16, 16)]
            key = _key_of(x)
            b1 = jnp.right_shift(key, 22) + 512
            plsc.addupdate_scatter(hist_v, [b1 * 16 + iota], ones)
            return 0

        lax.fori_loop(0, NV, hist_body, 0)
        _merge_hists(w, hist_v, merged_v, sh_hist, sh_merged)
        b1star, cbefore1 = _find_crossing(merged_v, Q1)

        def compact_body(i, cnt):
            x = data_v[pl.ds(i * 16, 16)]
            key = _key_of(x)
            b1 = jnp.right_shift(key, 22) + 512
            gidx = w * PW + i * 16 + iota
            m = (b1 >= b1star) & (gidx < N_DATA)
            rcs = plsc.cumsum(m.astype(jnp.int32))
            pos = cnt + rcs - 1
            m2 = m & (pos < CAP_L)
            plsc.store_scatter(ck_v, [pos], key, mask=m2)
            plsc.store_scatter(cg_v, [pos], gidx, mask=m2)
            return cnt + jnp.max(rcs)

        cnt = lax.fori_loop(0, NV, compact_body, jnp.int32(0))
        cnt = jnp.minimum(cnt, CAP_L)

        _zero_hist(hist_v)

        def hist2_body(i, _):
            kv = plsc.load_gather(ck_v, [i * 16 + iota])
            valid = (i * 16 + iota) < cnt
            in_bucket = (jnp.right_shift(kv, 22) + 512) == b1star
            b2 = jnp.bitwise_and(jnp.right_shift(kv, 12), NB - 1)
            plsc.addupdate_scatter(hist_v, [b2 * 16 + iota], ones,
                                   mask=valid & in_bucket)
            return 0

        lax.fori_loop(0, (cnt + 15) // 16, hist2_body, 0)
        _merge_hists(w, hist_v, merged_v, sh_hist, sh_merged)
        b2star, _cb2 = _find_crossing(merged_v, Q1 - cbefore1)
        t22 = (b1star - 512) * NB + b2star

        def filt_body(i, fcnt):
            kv = plsc.load_gather(ck_v, [i * 16 + iota])
            gv = plsc.load_gather(cg_v, [i * 16 + iota])
            valid = (i * 16 + iota) < cnt
            m = valid & (jnp.right_shift(kv, 12) >= t22)
            rcs = plsc.cumsum(m.astype(jnp.int32))
            pos = fcnt + rcs - 1
            m2 = m & (pos < CAP_F)
            plsc.store_scatter(fck_v, [pos], kv, mask=m2)
            plsc.store_scatter(fcg_v, [pos], gv, mask=m2)
            return fcnt + jnp.max(rcs)

        fcnt = lax.fori_loop(0, (cnt + 15) // 16, filt_body, jnp.int32(0))
        fcnt = jnp.minimum(fcnt, CAP_F)
        pltpu.sync_copy(fck_v, sh_fck.at[pl.ds(w * CAP_F, CAP_F)])
        pltpu.sync_copy(fcg_v, sh_fcg.at[pl.ds(w * CAP_F, CAP_F)])
        v16[...] = jnp.full((16,), fcnt, jnp.int32)
        pltpu.sync_copy(v16, sh_fcnt.at[pl.ds(w * 16, 16)])
        plsc.subcore_barrier()

        @pl.when(w == 0)
        def _sort():
            pltpu.sync_copy(sh_fck, allk_v)
            pltpu.sync_copy(sh_fcg, allg_v)
            pltpu.sync_copy(sh_fcnt, cnts_v)
            neg = jnp.full((16,), INT_MIN, jnp.int32)
            zer = jnp.zeros((16,), jnp.int32)
            for i in range(SRT // 16):
                ka[pl.ds(i * 16, 16)] = neg
                ga[pl.ds(i * 16, 16)] = zer

            tail = jnp.int32(0)
            for w2 in range(NW):
                cw = jnp.max(cnts_v[pl.ds(w2 * 16, 16)])

                def asm_body(v, tl, w2=w2, cw=cw):
                    kv = plsc.load_gather(allk_v, [w2 * CAP_F + v * 16 + iota])
                    gv = plsc.load_gather(allg_v, [w2 * CAP_F + v * 16 + iota])
                    m = (v * 16 + iota) < cw
                    pos = tl + v * 16 + iota
                    m = m & (pos < SRT)
                    plsc.store_scatter(ka, [pos], kv, mask=m)
                    plsc.store_scatter(ga, [pos], gv, mask=m)
                    return tl

                lax.fori_loop(0, (cw + 15) // 16, asm_body, tail)
                tail = jnp.minimum(tail + cw, SRT)

            src_k, src_g, dst_k, dst_g = ka, ga, kb, gb
            for p in range(5):
                shift = 7 * p
                def zh(i, _):
                    plsc.store_scatter(hist_v, [i * 16 + iota], zer)
                    return 0

                lax.fori_loop(0, 128, zh, 0)

                def ph_a(i, _, src_k=src_k, shift=shift):
                    kv = plsc.load_gather(src_k, [i * 16 + iota])
                    uk = jnp.bitwise_xor(kv, INT_MIN)
                    d = 127 - jnp.bitwise_and(
                        lax.shift_right_logical(uk, shift), 127)
                    plsc.addupdate_scatter(hist_v, [d * 16 + iota], ones)
                    return 0

                lax.fori_loop(0, SRT // 16, ph_a, 0)

                run = jnp.int32(0)
                for g in range(8):
                    acc = zer
                    for l in range(16):
                        acc = acc + plsc.load_gather(
                            hist_v, [(g * 16 + iota) * 16 + l])
                    cs = plsc.cumsum(acc)
                    offs_v[pl.ds(g * 16, 16)] = run + cs - acc
                    run = run + jnp.max(cs)

                def ph_b(i, _, src_k=src_k, src_g=src_g, dst_k=dst_k,
                         dst_g=dst_g, shift=shift):
                    kv = plsc.load_gather(src_k, [i * 16 + iota])
                    gv = plsc.load_gather(src_g, [i * 16 + iota])
                    uk = jnp.bitwise_xor(kv, INT_MIN)
                    d = 127 - jnp.bitwise_and(
                        lax.shift_right_logical(uk, shift), 127)
                    comp = d * 16 + iota
                    sd, _sv = plsc.sort_key_val(comp, iota)
                    dso = jnp.right_shift(sd, 4)
                    lo = jnp.bitwise_and(sd, 15)
                    prev = dso.at[jnp.maximum(iota - 1, 0)].get(
                        mode="promise_in_bounds")
                    isstart = (iota == 0) | (dso != prev)
                    stl = plsc.cummax(jnp.where(isstart, iota, 0))
                    rc = iota - stl
                    nxt = dso.at[jnp.minimum(iota + 1, 15)].get(
                        mode="promise_in_bounds")
                    islast = (iota == 15) | (dso != nxt)
                    base = plsc.load_gather(offs_v, [dso])
                    pos = base + rc
                    kvs = kv.at[lo].get(mode="promise_in_bounds")
                    gvs = gv.at[lo].get(mode="promise_in_bounds")
                    plsc.store_scatter(dst_k, [pos], kvs)
                    plsc.store_scatter(dst_g, [pos], gvs)
                    plsc.addupdate_scatter(offs_v, [dso], rc + 1, mask=islast)
                    return 0

                lax.fori_loop(0, SRT // 16, ph_b, 0)
                src_k, src_g, dst_k, dst_g = dst_k, dst_g, src_k, src_g

            pltpu.sync_copy(src_g.at[pl.ds(0, SEL)], sh_sel)

        plsc.subcore_barrier()

        pltpu.sync_copy(sh_sel.at[pl.ds(w * 64, 64)], sel_v)
        pltpu.async_copy(xs_hbm.at[sel_v], rx_v, sem).wait()
        pltpu.async_copy(ts_hbm.at[sel_v], rt_v, sem).wait()
        pltpu.sync_copy(rx_v, outx_hbm.at[pl.ds(N_OLD + w * 64, 64)])
        pltpu.sync_copy(rt_v, outt_hbm.at[pl.ds(N_OLD + w * 64, 64)])


def kernel(loss, x_f, t_f, x_s, t_s):
    loss_p = jnp.concatenate(
        [loss.reshape(-1),
         jnp.full((N_PAD - N_DATA,), -jnp.inf, jnp.float32)])
    out_x, out_t = _sc_topk(loss_p, x_f.reshape(-1), t_f.reshape(-1),
                            x_s.reshape(-1), t_s.reshape(-1))
    return (out_x[:N_OLD + M_TOP].reshape(-1, 1),
            out_t[:N_OLD + M_TOP].reshape(-1, 1))


# speedup vs baseline: 1.1920x; 1.0090x over previous
import functools
import jax
import jax.numpy as jnp
from jax import lax
from jax.experimental import pallas as pl
from jax.experimental.pallas import tpu as pltpu
from jax.experimental.pallas import tpu_sc as plsc

N_DATA = 1_000_000
N_OLD = 100_000
M_TOP = 1000
NW = 16
PW = 62_528
NV = PW // 16
N_PAD = NW * PW
NB = 1024
CAP_L = 512
CAP_F = 256
SRT = 1536
SEL = 1024
OUT_PAD = N_OLD + SEL
INT_MIN = jnp.int32(-2147483648)
Q1 = N_PAD - M_TOP

_mesh = plsc.VectorSubcoreMesh(core_axis_name="c", subcore_axis_name="s")


def _key_of(x):
    s = plsc.bitcast(x, jnp.int32)
    return jnp.where(s >= 0, s, INT_MIN - s)


def _butterfly_sum(v):
    iota = lax.iota(jnp.int32, 16)
    for k in (8, 4, 2, 1):
        v = v + v.at[jnp.bitwise_xor(iota, k)].get(mode="promise_in_bounds")
    return v


def _zero_hist(hist):
    iota = lax.iota(jnp.int32, 16)
    zeros = jnp.zeros((16,), jnp.int32)

    def body(i, _):
        plsc.store_scatter(hist, [i * 16 + iota], zeros)
        return 0

    lax.fori_loop(0, NB, body, 0)


def _merge_hists(w, hist, merged, sh_hist, sh_merged):
    iota = lax.iota(jnp.int32, 16)
    pltpu.sync_copy(hist, sh_hist.at[pl.ds(w * (NB * 16), NB * 16)])
    plsc.subcore_barrier()
    for r in range(NW):
        pltpu.sync_copy(
            sh_hist.at[pl.ds(r * (NB * 16) + w * NB, NB)],
            hist.at[pl.ds(r * NB, NB)],
        )

    def vsum(i, _):
        acc = jnp.zeros((16,), jnp.int32)
        for r in range(NW):
            acc = acc + plsc.load_gather(hist, [r * NB + i * 16 + iota])
        plsc.store_scatter(hist, [i * 16 + iota], acc)
        return 0

    lax.fori_loop(0, NB // 16, vsum, 0)

    def lsum(b, _):
        v = plsc.load_gather(hist, [b * 16 + iota])
        tot = _butterfly_sum(v)
        plsc.store_scatter(merged, [jnp.full((16,), b, jnp.int32)], tot,
                           mask=iota == 0)
        return 0

    lax.fori_loop(0, NB // 16, lsum, 0)
    pltpu.sync_copy(merged.at[pl.ds(0, NB // 16)], sh_merged.at[pl.ds(w * (NB // 16), NB // 16)])
    plsc.subcore_barrier()
    pltpu.sync_copy(sh_merged, merged)


def _find_crossing(merged, q):
    iota = lax.iota(jnp.int32, 16)

    def body(v, carry):
        run, bstar, cbefore = carry
        h = plsc.load_gather(merged, [v * 16 + iota])
        cs = plsc.cumsum(h)
        tot = jnp.max(cs)
        crossed = (run + cs) > q
        ffs = jnp.max(plsc.all_reduce_ffs(crossed).astype(jnp.int32))
        found = (bstar < 0) & (ffs < 16)
        excl = cs - h
        cb = run + jnp.max(jnp.where(iota == ffs, excl, 0))
        bstar = jnp.where(found, v * 16 + ffs, bstar)
        cbefore = jnp.where(found, cb, cbefore)
        return run + tot, bstar, cbefore

    _, bstar, cbefore = lax.fori_loop(
        0, NB // 16, body, (jnp.int32(0), jnp.int32(-1), jnp.int32(0)))
    return bstar, cbefore


@functools.partial(
    pl.kernel,
    out_type=[
        jax.ShapeDtypeStruct((OUT_PAD,), jnp.float32),
        jax.ShapeDtypeStruct((OUT_PAD,), jnp.float32),
    ],
    mesh=_mesh,
    compiler_params=pltpu.CompilerParams(needs_layout_passes=False),
    scratch_types=[
        pltpu.VMEM((PW,), jnp.float32),
        pltpu.VMEM((NB * 16,), jnp.int32),
        pltpu.VMEM((NB,), jnp.int32),
        pltpu.VMEM((CAP_L,), jnp.int32),
        pltpu.VMEM((CAP_L,), jnp.int32),
        pltpu.VMEM((CAP_F,), jnp.int32),
        pltpu.VMEM((CAP_F,), jnp.int32),
        pltpu.VMEM((NW * 16,), jnp.int32),
        pltpu.VMEM((NW * CAP_F,), jnp.int32),
        pltpu.VMEM((NW * CAP_F,), jnp.int32),
        pltpu.VMEM((SRT,), jnp.int32),
        pltpu.VMEM((SRT,), jnp.int32),
        pltpu.VMEM((SRT,), jnp.int32),
        pltpu.VMEM((SRT,), jnp.int32),
        pltpu.VMEM((128,), jnp.int32),
        pltpu.VMEM((64,), jnp.int32),
        pltpu.VMEM((64,), jnp.float32),
        pltpu.VMEM((64,), jnp.float32),
        pltpu.VMEM((16,), jnp.int32),
        pltpu.VMEM_SHARED((NW * NB * 16,), jnp.int32),
        pltpu.VMEM_SHARED((NB,), jnp.int32),
        pltpu.VMEM_SHARED((NW * CAP_F,), jnp.int32),
        pltpu.VMEM_SHARED((NW * CAP_F,), jnp.int32),
        pltpu.VMEM_SHARED((NW * 16,), jnp.int32),
        pltpu.VMEM_SHARED((SEL,), jnp.int32),
        pltpu.SemaphoreType.DMA,
    ],
)
def _sc_topk(loss_hbm, xf_hbm, tf_hbm, xs_hbm, ts_hbm, outx_hbm, outt_hbm,
             data_v, hist_v, merged_v, ck_v, cg_v, fck_v, fcg_v, cnts_v,
             allk_v, allg_v, ka, ga, kb, gb, offs_v, sel_v, rx_v, rt_v, v16,
             sh_hist, sh_merged, sh_fck, sh_fcg, sh_fcnt, sh_sel, sem):
    c = lax.axis_index("c")
    w = lax.axis_index("s")
    iota = lax.iota(jnp.int32, 16)
    ones = jnp.ones((16,), jnp.int32)

    @pl.when(c == 1)
    def _copy_old():
        @pl.when(w < 12)
        def _():
            a = w * 6248
            pltpu.sync_copy(xf_hbm.at[pl.ds(a, 6248)], data_v.at[pl.ds(0, 6248)])
            pltpu.sync_copy(data_v.at[pl.ds(0, 6248)], outx_hbm.at[pl.ds(a, 6248)])
            pltpu.sync_copy(tf_hbm.at[pl.ds(a, 6248)], data_v.at[pl.ds(0, 6248)])
            pltpu.sync_copy(data_v.at[pl.ds(0, 6248)], outt_hbm.at[pl.ds(a, 6248)])

        @pl.when(w >= 12)
        def _():
            a = 74976 + (w - 12) * 6256
            pltpu.sync_copy(xf_hbm.at[pl.ds(a, 6256)], data_v.at[pl.ds(0, 6256)])
            pltpu.sync_copy(data_v.at[pl.ds(0, 6256)], outx_hbm.at[pl.ds(a, 6256)])
            pltpu.sync_copy(tf_hbm.at[pl.ds(a, 6256)], data_v.at[pl.ds(0, 6256)])
            pltpu.sync_copy(data_v.at[pl.ds(0, 6256)], outt_hbm.at[pl.ds(a, 6256)])

    @pl.when(c == 0)
    def _topk():
        pltpu.sync_copy(loss_hbm.at[pl.ds(w * PW, PW)], data_v)
        _zero_hist(hist_v)

        def hist_body(i, _):
            for u in range(4):
                x = data_v[pl.ds((i * 4 + u) * 16, 16)]
                key = _key_of(x)
                b1 = jnp.right_shift(key, 22) + 512
                plsc.addupdate_scatter(hist_v, [b1 * 16 + iota], ones)
            return 0

        lax.fori_loop(0, NV // 4, hist_body, 0)
        _merge_hists(w, hist_v, merged_v, sh_hist, sh_merged)
        b1star, cbefore1 = _find_crossing(merged_v, Q1)

        def compact_body(i, cnt):
            x = data_v[pl.ds(i * 16, 16)]
            key = _key_of(x)
            b1 = jnp.right_shift(key, 22) + 512
            gidx = w * PW + i * 16 + iota
            m = (b1 >= b1star) & (gidx < N_DATA)
            rcs = plsc.cumsum(m.astype(jnp.int32))
            pos = cnt + rcs - 1
            m2 = m & (pos < CAP_L)
            plsc.store_scatter(ck_v, [pos], key, mask=m2)
            plsc.store_scatter(cg_v, [pos], gidx, mask=m2)
            return cnt + jnp.max(rcs)

        cnt = lax.fori_loop(0, NV, compact_body, jnp.int32(0))
        cnt = jnp.minimum(cnt, CAP_L)

        _zero_hist(hist_v)

        def hist2_body(i, _):
            kv = plsc.load_gather(ck_v, [i * 16 + iota])
            valid = (i * 16 + iota) < cnt
            in_bucket = (jnp.right_shift(kv, 22) + 512) == b1star
            b2 = jnp.bitwise_and(jnp.right_shift(kv, 12), NB - 1)
            plsc.addupdate_scatter(hist_v, [b2 * 16 + iota], ones,
                                   mask=valid & in_bucket)
            return 0

        lax.fori_loop(0, (cnt + 15) // 16, hist2_body, 0)
        _merge_hists(w, hist_v, merged_v, sh_hist, sh_merged)
        b2star, _cb2 = _find_crossing(merged_v, Q1 - cbefore1)
        t22 = (b1star - 512) * NB + b2star

        def filt_body(i, fcnt):
            kv = plsc.load_gather(ck_v, [i * 16 + iota])
            gv = plsc.load_gather(cg_v, [i * 16 + iota])
            valid = (i * 16 + iota) < cnt
            m = valid & (jnp.right_shift(kv, 12) >= t22)
            rcs = plsc.cumsum(m.astype(jnp.int32))
            pos = fcnt + rcs - 1
            m2 = m & (pos < CAP_F)
            plsc.store_scatter(fck_v, [pos], kv, mask=m2)
            plsc.store_scatter(fcg_v, [pos], gv, mask=m2)
            return fcnt + jnp.max(rcs)

        fcnt = lax.fori_loop(0, (cnt + 15) // 16, filt_body, jnp.int32(0))
        fcnt = jnp.minimum(fcnt, CAP_F)
        pltpu.sync_copy(fck_v, sh_fck.at[pl.ds(w * CAP_F, CAP_F)])
        pltpu.sync_copy(fcg_v, sh_fcg.at[pl.ds(w * CAP_F, CAP_F)])
        v16[...] = jnp.full((16,), fcnt, jnp.int32)
        pltpu.sync_copy(v16, sh_fcnt.at[pl.ds(w * 16, 16)])
        plsc.subcore_barrier()

        @pl.when(w == 0)
        def _sort():
            pltpu.sync_copy(sh_fck, allk_v)
            pltpu.sync_copy(sh_fcg, allg_v)
            pltpu.sync_copy(sh_fcnt, cnts_v)
            neg = jnp.full((16,), INT_MIN, jnp.int32)
            zer = jnp.zeros((16,), jnp.int32)
            for i in range(SRT // 16):
                ka[pl.ds(i * 16, 16)] = neg
                ga[pl.ds(i * 16, 16)] = zer

            tail = jnp.int32(0)
            for w2 in range(NW):
                cw = jnp.max(cnts_v[pl.ds(w2 * 16, 16)])

                def asm_body(v, tl, w2=w2, cw=cw):
                    kv = plsc.load_gather(allk_v, [w2 * CAP_F + v * 16 + iota])
                    gv = plsc.load_gather(allg_v, [w2 * CAP_F + v * 16 + iota])
                    m = (v * 16 + iota) < cw
                    pos = tl + v * 16 + iota
                    m = m & (pos < SRT)
                    plsc.store_scatter(ka, [pos], kv, mask=m)
                    plsc.store_scatter(ga, [pos], gv, mask=m)
                    return tl

                lax.fori_loop(0, (cw + 15) // 16, asm_body, tail)
                tail = jnp.minimum(tail + cw, SRT)

            src_k, src_g, dst_k, dst_g = ka, ga, kb, gb
            for p in range(5):
                shift = 7 * p
                def zh(i, _):
                    plsc.store_scatter(hist_v, [i * 16 + iota], zer)
                    return 0

                lax.fori_loop(0, 128, zh, 0)

                def ph_a(i, _, src_k=src_k, shift=shift):
                    kv = plsc.load_gather(src_k, [i * 16 + iota])
                    uk = jnp.bitwise_xor(kv, INT_MIN)
                    d = 127 - jnp.bitwise_and(
                        lax.shift_right_logical(uk, shift), 127)
                    plsc.addupdate_scatter(hist_v, [d * 16 + iota], ones)
                    return 0

                lax.fori_loop(0, SRT // 16, ph_a, 0)

                run = jnp.int32(0)
                for g in range(8):
                    acc = zer
                    for l in range(16):
                        acc = acc + plsc.load_gather(
                            hist_v, [(g * 16 + iota) * 16 + l])
                    cs = plsc.cumsum(acc)
                    offs_v[pl.ds(g * 16, 16)] = run + cs - acc
                    run = run + jnp.max(cs)

                def ph_b(i, _, src_k=src_k, src_g=src_g, dst_k=dst_k,
                         dst_g=dst_g, shift=shift):
                    kv = plsc.load_gather(src_k, [i * 16 + iota])
                    gv = plsc.load_gather(src_g, [i * 16 + iota])
                    uk = jnp.bitwise_xor(kv, INT_MIN)
                    d = 127 - jnp.bitwise_and(
                        lax.shift_right_logical(uk, shift), 127)
                    comp = d * 16 + iota
                    sd, _sv = plsc.sort_key_val(comp, iota)
                    dso = jnp.right_shift(sd, 4)
                    lo = jnp.bitwise_and(sd, 15)
                    prev = dso.at[jnp.maximum(iota - 1, 0)].get(
                        mode="promise_in_bounds")
                    isstart = (iota == 0) | (dso != prev)
                    stl = plsc.cummax(jnp.where(isstart, iota, 0))
                    rc = iota - stl
                    nxt = dso.at[jnp.minimum(iota + 1, 15)].get(
                        mode="promise_in_bounds")
                    islast = (iota == 15) | (dso != nxt)
                    base = plsc.load_gather(offs_v, [dso])
                    pos = base + rc
                    kvs = kv.at[lo].get(mode="promise_in_bounds")
                    gvs = gv.at[lo].get(mode="promise_in_bounds")
                    plsc.store_scatter(dst_k, [pos], kvs)
                    plsc.store_scatter(dst_g, [pos], gvs)
                    plsc.addupdate_scatter(offs_v, [dso], rc + 1, mask=islast)
                    return 0

                lax.fori_loop(0, SRT // 16, ph_b, 0)
                src_k, src_g, dst_k, dst_g = dst_k, dst_g, src_k, src_g

            pltpu.sync_copy(src_g.at[pl.ds(0, SEL)], sh_sel)

        plsc.subcore_barrier()

        pltpu.sync_copy(sh_sel.at[pl.ds(w * 64, 64)], sel_v)
        pltpu.async_copy(xs_hbm.at[sel_v], rx_v, sem).wait()
        pltpu.async_copy(ts_hbm.at[sel_v], rt_v, sem).wait()
        pltpu.sync_copy(rx_v, outx_hbm.at[pl.ds(N_OLD + w * 64, 64)])
        pltpu.sync_copy(rt_v, outt_hbm.at[pl.ds(N_OLD + w * 64, 64)])


def kernel(loss, x_f, t_f, x_s, t_s):
    loss_p = jnp.concatenate(
        [loss.reshape(-1),
         jnp.full((N_PAD - N_DATA,), -jnp.inf, jnp.float32)])
    out_x, out_t = _sc_topk(loss_p, x_f.reshape(-1), t_f.reshape(-1),
                            x_s.reshape(-1), t_s.reshape(-1))
    return (out_x[:N_OLD + M_TOP].reshape(-1, 1),
            out_t[:N_OLD + M_TOP].reshape(-1, 1))
